# Initial kernel scaffold; baseline (speedup 1.0000x reference)
import functools

import jax
import jax.numpy as jnp
from jax import lax
from jax.experimental import pallas as pl
from jax.experimental.pallas import tpu as pltpu
from jax.experimental.pallas import tpu_sc as plsc

D = 32
NC, NS = 2, 16
NW = NC * NS


@functools.partial(jax.jit, static_argnums=(2, 3))
def _gather(idx_flat, table, b_total, chunk):
    b_per_w = b_total // NW
    nchunk = b_per_w // chunk
    mesh = plsc.VectorSubcoreMesh(
        core_axis_name="c", subcore_axis_name="s",
        num_cores=NC, num_subcores=NS,
    )

    @functools.partial(
        pl.kernel,
        out_type=jax.ShapeDtypeStruct((b_total, D), jnp.float32),
        mesh=mesh,
        scratch_types=[
            pltpu.VMEM((chunk,), jnp.int32),
            pltpu.VMEM((chunk, D), jnp.float32),
            pltpu.SemaphoreType.DMA,
        ],
    )
    def k(idx_hbm, table_hbm, out_hbm, idx_v, rows_v, sem):
        wid = lax.axis_index("s") * NC + lax.axis_index("c")
        base = wid * b_per_w

        def body(c, carry):
            start = base + c * chunk
            pltpu.sync_copy(idx_hbm.at[pl.ds(start, chunk)], idx_v)
            pltpu.async_copy(table_hbm.at[idx_v], rows_v, sem).wait()
            pltpu.sync_copy(rows_v, out_hbm.at[pl.ds(start, chunk)])
            return carry

        lax.fori_loop(0, nchunk, body, 0)

    return k(idx_flat, table)


def kernel(x, emb_weight):
    b, h = x.shape
    flat = x.reshape(b * h)
    out = _gather(flat, emb_weight, b * h, 1024)
    return out.reshape(b, h, D)


# speedup vs baseline: 1.0945x; 1.0945x over previous
import functools

import jax
import jax.numpy as jnp
from jax import lax
from jax.experimental import pallas as pl
from jax.experimental.pallas import tpu as pltpu
from jax.experimental.pallas import tpu_sc as plsc

D = 32
NC, NS = 2, 16
NW = NC * NS


@functools.partial(jax.jit, static_argnums=(2, 3))
def _gather(idx_flat, table, b_total, chunk):
    b_per_w = b_total // NW
    nchunk = b_per_w // chunk
    mesh = plsc.VectorSubcoreMesh(
        core_axis_name="c", subcore_axis_name="s",
        num_cores=NC, num_subcores=NS,
    )

    @functools.partial(
        pl.kernel,
        out_type=jax.ShapeDtypeStruct((b_total, D), jnp.float32),
        mesh=mesh,
        scratch_types=[
            pltpu.VMEM((chunk,), jnp.int32),
            pltpu.VMEM((chunk, D), jnp.float32),
            pltpu.SemaphoreType.DMA,
        ],
        compiler_params=pltpu.CompilerParams(use_tc_tiling_on_sc=False),
    )
    def k(idx_hbm, table_hbm, out_hbm, idx_v, rows_v, sem):
        wid = lax.axis_index("s") * NC + lax.axis_index("c")
        base = wid * b_per_w

        def body(c, carry):
            start = base + c * chunk
            pltpu.sync_copy(idx_hbm.at[pl.ds(start, chunk)], idx_v)
            pltpu.async_copy(table_hbm.at[idx_v], rows_v, sem).wait()
            pltpu.sync_copy(rows_v, out_hbm.at[pl.ds(start, chunk)])
            return carry

        lax.fori_loop(0, nchunk, body, 0)

    return k(idx_flat, table)


def kernel(x, emb_weight):
    b, h = x.shape
    flat = x.reshape(b * h)
    out = _gather(flat, emb_weight, b * h, 1024)
    return out.reshape(b, h, D)


# speedup vs baseline: 1.1095x; 1.0137x over previous
import functools

import jax
import jax.numpy as jnp
from jax import lax
from jax.experimental import pallas as pl
from jax.experimental.pallas import tpu as pltpu
from jax.experimental.pallas import tpu_sc as plsc

D = 32
NC, NS = 2, 16
NW = NC * NS


@functools.partial(jax.jit, static_argnums=(2, 3))
def _gather(idx_flat, table, b_total, chunk):
    b_per_w = b_total // NW
    nchunk = b_per_w // chunk
    assert nchunk % 2 == 0 and nchunk >= 4
    mesh = plsc.VectorSubcoreMesh(
        core_axis_name="c", subcore_axis_name="s",
        num_cores=NC, num_subcores=NS,
    )

    @functools.partial(
        pl.kernel,
        out_type=jax.ShapeDtypeStruct((b_total, D), jnp.float32),
        mesh=mesh,
        scratch_types=[
            pltpu.VMEM((chunk,), jnp.int32),
            pltpu.VMEM((chunk,), jnp.int32),
            pltpu.VMEM((chunk, D), jnp.float32),
            pltpu.VMEM((chunk, D), jnp.float32),
            pltpu.SemaphoreType.DMA,
            pltpu.SemaphoreType.DMA,
            pltpu.SemaphoreType.DMA,
            pltpu.SemaphoreType.DMA,
            pltpu.SemaphoreType.DMA,
            pltpu.SemaphoreType.DMA,
        ],
        compiler_params=pltpu.CompilerParams(use_tc_tiling_on_sc=False),
    )
    def k(idx_hbm, table_hbm, out_hbm, idx0, idx1, rows0, rows1,
          si0, si1, sg0, sg1, ss0, ss1):
        wid = lax.axis_index("s") * NC + lax.axis_index("c")
        base = wid * b_per_w
        idx_b = (idx0, idx1)
        rows_b = (rows0, rows1)
        sem_i = (si0, si1)
        sem_g = (sg0, sg1)
        sem_s = (ss0, ss1)

        def istart(c, b):
            pltpu.async_copy(
                idx_hbm.at[pl.ds(base + c * chunk, chunk)], idx_b[b],
                sem_i[b])

        istart(0, 0)
        istart(1, 1)

        def body(g, carry):
            for b in range(2):
                c = 2 * g + b
                pltpu.make_async_copy(
                    idx_hbm.at[pl.ds(base + c * chunk, chunk)], idx_b[b],
                    sem_i[b]).wait()

                @pl.when(g > 0)
                def _():
                    pltpu.make_async_copy(
                        rows_b[b],
                        out_hbm.at[pl.ds(base + (c - 2) * chunk, chunk)],
                        sem_s[b]).wait()

                pltpu.async_copy(table_hbm.at[idx_b[b]], rows_b[b], sem_g[b])
            for b in range(2):
                c = 2 * g + b
                pltpu.make_async_copy(
                    table_hbm.at[idx_b[b]], rows_b[b], sem_g[b]).wait()
                pltpu.async_copy(
                    rows_b[b],
                    out_hbm.at[pl.ds(base + c * chunk, chunk)], sem_s[b])

                @pl.when(c + 2 < nchunk)
                def _():
                    istart(c + 2, b)
            return carry

        lax.fori_loop(0, nchunk // 2, body, 0)

        for b in range(2):
            c = nchunk - 2 + b
            pltpu.make_async_copy(
                rows_b[b],
                out_hbm.at[pl.ds(base + c * chunk, chunk)], sem_s[b]).wait()

    return k(idx_flat, table)


def kernel(x, emb_weight):
    b, h = x.shape
    flat = x.reshape(b * h)
    out = _gather(flat, emb_weight, b * h, 1600)
    return out.reshape(b, h, D)
